# Initial kernel scaffold; baseline (speedup 1.0000x reference)
import jax
import jax.numpy as jnp
from jax.experimental import pallas as pl


def kernel(x):
    raise NotImplementedError("write your pallas kernel here")


# speedup vs baseline: 2.6119x; 2.6119x over previous
import functools

import jax
import jax.numpy as jnp
from jax.experimental import pallas as pl
from jax.experimental.pallas import tpu as pltpu


def _round_up(n, m):
    return ((n + m - 1) // m) * m


def _mean_std_block_kernel(t_real, x_ref, out_ref):
    x = x_ref[...].astype(jnp.float32)
    s = jnp.sum(x, axis=-1, keepdims=True)
    sq = jnp.sum(x * x, axis=-1, keepdims=True)
    mean = s * (1.0 / t_real)
    var = jnp.maximum(sq - s * mean, 0.0) * (1.0 / (t_real - 1))
    std = jnp.sqrt(var)
    out_ref[:, 0:1] = mean.astype(out_ref.dtype)
    out_ref[:, 1:2] = std.astype(out_ref.dtype)


def kernel(x):
    B, D, T = x.shape
    R = B * D
    x2 = x.reshape(R, T)

    itemsize = jnp.dtype(x.dtype).itemsize
    T_pad = _round_up(T, 128)
    cap_rows = max(8, (2 * 1024 * 1024) // (T_pad * itemsize) // 8 * 8)
    tile_r = min(cap_rows, 2048, _round_up(R, 8))
    R_pad = _round_up(R, tile_r)

    if (R_pad, T_pad) != (R, T):
        x2 = jnp.pad(x2, ((0, R_pad - R), (0, T_pad - T)))

    out = pl.pallas_call(
        functools.partial(_mean_std_block_kernel, T),
        out_shape=jax.ShapeDtypeStruct((R_pad, 2), x.dtype),
        grid=(R_pad // tile_r,),
        in_specs=[pl.BlockSpec((tile_r, T_pad), lambda i: (i, 0))],
        out_specs=pl.BlockSpec((tile_r, 2), lambda i: (i, 0)),
        compiler_params=pltpu.CompilerParams(
            dimension_semantics=("parallel",)
        ),
    )(x2)

    mean = out[:R, 0].reshape(B, D)
    std = out[:R, 1].reshape(B, D)
    return jnp.concatenate([mean, std], axis=1)


# speedup vs baseline: 3.1027x; 1.1879x over previous
import functools

import jax
import jax.numpy as jnp
from jax.experimental import pallas as pl
from jax.experimental.pallas import tpu as pltpu


def _round_up(n, m):
    return ((n + m - 1) // m) * m


def _mean_std_block_kernel(t_real, x_ref, out_ref):
    x = x_ref[...].astype(jnp.float32)
    s = jnp.sum(x, axis=-1, keepdims=True)
    sq = jnp.sum(x * x, axis=-1, keepdims=True)
    mean = s * (1.0 / t_real)
    var = jnp.maximum(sq - s * mean, 0.0) * (1.0 / (t_real - 1))
    std = jnp.sqrt(var)
    out_ref[:, 0:1] = mean.astype(out_ref.dtype)
    out_ref[:, 1:2] = std.astype(out_ref.dtype)


def kernel(x):
    B, D, T = x.shape
    R = B * D
    x2 = x.reshape(R, T)

    itemsize = jnp.dtype(x.dtype).itemsize
    T_pad = _round_up(T, 128)
    cap_rows = max(8, (8 * 1024 * 1024) // (T_pad * itemsize) // 8 * 8)
    tile_r = min(cap_rows, _round_up(R, 8))
    R_pad = _round_up(R, tile_r)

    if (R_pad, T_pad) != (R, T):
        x2 = jnp.pad(x2, ((0, R_pad - R), (0, T_pad - T)))

    out = pl.pallas_call(
        functools.partial(_mean_std_block_kernel, T),
        out_shape=jax.ShapeDtypeStruct((R_pad, 2), x.dtype),
        grid=(R_pad // tile_r,),
        in_specs=[pl.BlockSpec((tile_r, T_pad), lambda i: (i, 0))],
        out_specs=pl.BlockSpec((tile_r, 2), lambda i: (i, 0)),
        compiler_params=pltpu.CompilerParams(
            dimension_semantics=("parallel",)
        ),
    )(x2)

    mean = out[:R, 0].reshape(B, D)
    std = out[:R, 1].reshape(B, D)
    return jnp.concatenate([mean, std], axis=1)


# speedup vs baseline: 3.1670x; 1.0207x over previous
import functools

import jax
import jax.numpy as jnp
from jax.experimental import pallas as pl
from jax.experimental.pallas import tpu as pltpu


def _round_up(n, m):
    return ((n + m - 1) // m) * m


def _mean_std_block_kernel(t_real, x_ref, out_ref):
    x = x_ref[...].astype(jnp.float32)
    s = jnp.sum(x, axis=-1, keepdims=True)
    sq = jnp.sum(x * x, axis=-1, keepdims=True)
    mean = s * (1.0 / t_real)
    var = jnp.maximum(sq - s * mean, 0.0) * (1.0 / (t_real - 1))
    std = jnp.sqrt(var)
    out_ref[:, 0:1] = mean.astype(out_ref.dtype)
    out_ref[:, 1:2] = std.astype(out_ref.dtype)


def kernel(x):
    B, D, T = x.shape
    R = B * D
    x2 = x.reshape(R, T)

    itemsize = jnp.dtype(x.dtype).itemsize
    T_pad = _round_up(T, 128)
    cap_rows = max(8, (16 * 1024 * 1024) // (T_pad * itemsize) // 8 * 8)
    tile_r = min(cap_rows, _round_up(R, 8))
    R_pad = _round_up(R, tile_r)

    if (R_pad, T_pad) != (R, T):
        x2 = jnp.pad(x2, ((0, R_pad - R), (0, T_pad - T)))

    out = pl.pallas_call(
        functools.partial(_mean_std_block_kernel, T),
        out_shape=jax.ShapeDtypeStruct((R_pad, 2), x.dtype),
        grid=(R_pad // tile_r,),
        in_specs=[pl.BlockSpec((tile_r, T_pad), lambda i: (i, 0))],
        out_specs=pl.BlockSpec((tile_r, 2), lambda i: (i, 0)),
        compiler_params=pltpu.CompilerParams(
            dimension_semantics=("parallel",)
        ),
    )(x2)

    mean = out[:R, 0].reshape(B, D)
    std = out[:R, 1].reshape(B, D)
    return jnp.concatenate([mean, std], axis=1)


# speedup vs baseline: 3.6398x; 1.1493x over previous
import functools

import jax
import jax.numpy as jnp
from jax.experimental import pallas as pl
from jax.experimental.pallas import tpu as pltpu


def _round_up(n, m):
    return ((n + m - 1) // m) * m


def _mean_std_core(t_real, x, out_ref):
    s = jnp.sum(x, axis=-1, keepdims=True)
    sq = jnp.sum(x * x, axis=-1, keepdims=True)
    mean = s * (1.0 / t_real)
    var = jnp.maximum(sq - s * mean, 0.0) * (1.0 / (t_real - 1))
    std = jnp.sqrt(var)
    out_ref[:, 0:1] = mean.astype(out_ref.dtype)
    out_ref[:, 1:2] = std.astype(out_ref.dtype)


def _mean_std_block_kernel(t_real, xa_ref, xb_ref, oa_ref, ob_ref):
    _mean_std_core(t_real, xa_ref[...].astype(jnp.float32), oa_ref)
    _mean_std_core(t_real, xb_ref[...].astype(jnp.float32), ob_ref)


def kernel(x):
    B, D, T = x.shape
    R = B * D
    x2 = x.reshape(R, T)

    itemsize = jnp.dtype(x.dtype).itemsize
    T_pad = _round_up(T, 128)
    cap_rows = max(8, (8 * 1024 * 1024) // (T_pad * itemsize) // 8 * 8)
    tile_r = min(cap_rows, _round_up(R, 8))
    R_pad = _round_up(R, 2 * tile_r)

    if (R_pad, T_pad) != (R, T):
        x2 = jnp.pad(x2, ((0, R_pad - R), (0, T_pad - T)))

    n = R_pad // (2 * tile_r)
    half = jax.ShapeDtypeStruct((n * tile_r, 2), x.dtype)
    out_a, out_b = pl.pallas_call(
        functools.partial(_mean_std_block_kernel, T),
        out_shape=(half, half),
        grid=(n,),
        in_specs=[
            pl.BlockSpec((tile_r, T_pad), lambda i: (2 * i, 0)),
            pl.BlockSpec((tile_r, T_pad), lambda i: (2 * i + 1, 0)),
        ],
        out_specs=(
            pl.BlockSpec((tile_r, 2), lambda i: (i, 0)),
            pl.BlockSpec((tile_r, 2), lambda i: (i, 0)),
        ),
        compiler_params=pltpu.CompilerParams(
            dimension_semantics=("parallel",)
        ),
    )(x2, x2)

    out = jnp.stack(
        [out_a.reshape(n, tile_r, 2), out_b.reshape(n, tile_r, 2)], axis=1
    ).reshape(R_pad, 2)
    mean = out[:R, 0].reshape(B, D)
    std = out[:R, 1].reshape(B, D)
    return jnp.concatenate([mean, std], axis=1)
